# baseline (speedup 1.0000x reference)
import jax
import jax.numpy as jnp
from jax.experimental import pallas as pl
from jax.experimental.pallas import tpu as pltpu

PENALTY = 1.2
BV = 50048


def _damp_kernel(ids_col_ref, ids_row_ref, logits_ref, out_ref):
    S = ids_col_ref.shape[1]
    vb = pl.program_id(1)

    col = ids_col_ref[0]
    row = ids_row_ref[0]

    r = jax.lax.broadcasted_iota(jnp.int32, (S, S), 0)
    c = jax.lax.broadcasted_iota(jnp.int32, (S, S), 1)
    dup = jnp.any((col == row) & (c < r), axis=1, keepdims=True)
    ids_clean = jnp.where(dup, -1, col) - vb * BV

    vids = jax.lax.broadcasted_iota(jnp.int32, (S, BV), 1)
    oh = jnp.where(ids_clean == vids, 1.0, 0.0).astype(jnp.bfloat16)

    tril = jnp.where(c < r, 1.0, 0.0).astype(jnp.bfloat16)

    counts = jax.lax.dot(tril, oh, preferred_element_type=jnp.float32)

    x = logits_ref[0]
    cs = counts * jnp.float32(-(1.0 - 1.0 / PENALTY))
    out_ref[0] = x * cs + x


@jax.jit
def kernel(logits, input_ids):
    B, S, V = logits.shape
    ids_col = input_ids.reshape(B, S, 1)
    ids_row = input_ids.reshape(B, 1, S)
    return pl.pallas_call(
        _damp_kernel,
        grid=(B, pl.cdiv(V, BV)),
        in_specs=[
            pl.BlockSpec((1, S, 1), lambda b, v: (b, 0, 0)),
            pl.BlockSpec((1, 1, S), lambda b, v: (b, 0, 0)),
            pl.BlockSpec((1, S, BV), lambda b, v: (b, 0, v)),
        ],
        out_specs=pl.BlockSpec((1, S, BV), lambda b, v: (b, 0, v)),
        out_shape=jax.ShapeDtypeStruct((B, S, V), logits.dtype),
        compiler_params=pltpu.CompilerParams(
            dimension_semantics=("parallel", "parallel"),
        ),
    )(ids_col, ids_row, logits)


# speedup vs baseline: 1.0151x; 1.0151x over previous
import jax
import jax.numpy as jnp
from jax.experimental import pallas as pl
from jax.experimental.pallas import tpu as pltpu

PENALTY = 1.2
BV = 100096


def _damp_kernel(ids_col_ref, ids_row_ref, logits_ref, out_ref):
    S = ids_col_ref.shape[1]
    vb = pl.program_id(1)

    col = ids_col_ref[0]
    row = ids_row_ref[0]

    r = jax.lax.broadcasted_iota(jnp.int32, (S, S), 0)
    c = jax.lax.broadcasted_iota(jnp.int32, (S, S), 1)
    dup = jnp.any((col == row) & (c < r), axis=1, keepdims=True)
    ids_clean = jnp.where(dup, -1, col) - vb * BV

    vids = jax.lax.broadcasted_iota(jnp.int32, (S, BV), 1)
    oh = jnp.where(ids_clean == vids, 1.0, 0.0).astype(jnp.bfloat16)

    tril = jnp.where(c < r, 1.0, 0.0).astype(jnp.bfloat16)

    counts = jax.lax.dot(tril, oh, preferred_element_type=jnp.float32)

    x = logits_ref[0]
    cs = counts * jnp.float32(-(1.0 - 1.0 / PENALTY))
    out_ref[0] = x * cs + x


@jax.jit
def kernel(logits, input_ids):
    B, S, V = logits.shape
    ids_col = input_ids.reshape(B, S, 1)
    ids_row = input_ids.reshape(B, 1, S)
    return pl.pallas_call(
        _damp_kernel,
        grid=(B, pl.cdiv(V, BV)),
        in_specs=[
            pl.BlockSpec((1, S, 1), lambda b, v: (b, 0, 0)),
            pl.BlockSpec((1, 1, S), lambda b, v: (b, 0, 0)),
            pl.BlockSpec((1, S, BV), lambda b, v: (b, 0, v)),
        ],
        out_specs=pl.BlockSpec((1, S, BV), lambda b, v: (b, 0, v)),
        out_shape=jax.ShapeDtypeStruct((B, S, V), logits.dtype),
        compiler_params=pltpu.CompilerParams(
            dimension_semantics=("parallel", "parallel"),
            vmem_limit_bytes=120 * 1024 * 1024,
        ),
    )(ids_col, ids_row, logits)
